# baseline (speedup 1.0000x reference)
import functools

import jax
import jax.numpy as jnp
from jax import lax
from jax.experimental import pallas as pl
from jax.experimental.pallas import tpu as pltpu
from jax.experimental.pallas import tpu_sc as plsc

_NUM_CORES = 2
_NUM_SUBCORES = 16
_NUM_WORKERS = _NUM_CORES * _NUM_SUBCORES
_TC_ROWS = 5120
_TC_BLOCK = 512


@functools.lru_cache(maxsize=None)
def _make_sc_copy(row0: int, n_rows: int, dim: int):
    rows_per_w = n_rows // _NUM_WORKERS
    chunk = min(rows_per_w, 48)
    nchunk = rows_per_w // chunk
    mesh = plsc.VectorSubcoreMesh(core_axis_name="c", subcore_axis_name="s")

    @functools.partial(
        pl.kernel,
        mesh=mesh,
        out_type=jax.ShapeDtypeStruct((n_rows, dim), jnp.float32),
        scratch_types=[
            pltpu.VMEM((chunk, dim), jnp.float32),
            pltpu.VMEM((chunk, dim), jnp.float32),
            pltpu.SemaphoreType.DMA,
            pltpu.SemaphoreType.DMA,
            pltpu.SemaphoreType.DMA,
            pltpu.SemaphoreType.DMA,
        ],
    )
    def k(emb_hbm, out_hbm, buf0, buf1, rsem0, rsem1, wsem0, wsem1):
        bufs = (buf0, buf1)
        rsems = (rsem0, rsem1)
        wsems = (wsem0, wsem1)
        wid = lax.axis_index("s") * _NUM_CORES + lax.axis_index("c")
        base = wid * rows_per_w

        def read(c):
            b = c % 2
            return pltpu.async_copy(
                emb_hbm.at[pl.ds(row0 + base + c * chunk, chunk)],
                bufs[b], rsems[b])

        def write(c):
            b = c % 2
            return pltpu.async_copy(
                bufs[b], out_hbm.at[pl.ds(base + c * chunk, chunk)], wsems[b])

        reads = {0: read(0)}
        writes = {}
        for c in range(nchunk):
            if c + 1 < nchunk:
                if c - 1 >= 0:
                    writes.pop(c - 1).wait()
                reads[c + 1] = read(c + 1)
            reads.pop(c).wait()
            writes[c] = write(c)
        for w in writes.values():
            w.wait()

    return k


def _tc_body(in_ref, out_ref):
    out_ref[...] = in_ref[...]


@functools.lru_cache(maxsize=None)
def _make_tc_copy(n_rows: int, dim: int):
    return pl.pallas_call(
        _tc_body,
        grid=(n_rows // _TC_BLOCK,),
        in_specs=[pl.BlockSpec((_TC_BLOCK, dim), lambda i: (i, 0))],
        out_specs=pl.BlockSpec((_TC_BLOCK, dim), lambda i: (i, 0)),
        out_shape=jax.ShapeDtypeStruct((n_rows, dim), jnp.float32),
    )


def kernel(x, emb_weight):
    seq_len = x.shape[1]
    dim = emb_weight.shape[1]
    sc_rows = seq_len - _TC_ROWS
    sc_part = _make_sc_copy(_TC_ROWS, sc_rows, dim)(emb_weight)
    tc_part = _make_tc_copy(_TC_ROWS, dim)(emb_weight)
    return jnp.concatenate([tc_part, sc_part], axis=0)


# speedup vs baseline: 1.3925x; 1.3925x over previous
import functools

import jax
import jax.numpy as jnp
from jax import lax
from jax.experimental import pallas as pl
from jax.experimental.pallas import tpu as pltpu
from jax.experimental.pallas import tpu_sc as plsc

_NUM_CORES = 2
_NUM_SUBCORES = 16
_NUM_WORKERS = _NUM_CORES * _NUM_SUBCORES
_TC_ROWS = 5120
_TC_BLOCK = 512


@functools.lru_cache(maxsize=None)
def _make_sc_copy(row0: int, seq_len: int, dim: int):
    n_rows = seq_len - row0
    rows_per_w = n_rows // _NUM_WORKERS
    chunk = min(rows_per_w, 48)
    nchunk = rows_per_w // chunk
    mesh = plsc.VectorSubcoreMesh(core_axis_name="c", subcore_axis_name="s")

    @functools.partial(
        pl.kernel,
        mesh=mesh,
        out_type=jax.ShapeDtypeStruct((seq_len, dim), jnp.float32),
        scratch_types=[
            pltpu.VMEM((chunk, dim), jnp.float32),
            pltpu.VMEM((chunk, dim), jnp.float32),
            pltpu.SemaphoreType.DMA,
            pltpu.SemaphoreType.DMA,
            pltpu.SemaphoreType.DMA,
            pltpu.SemaphoreType.DMA,
        ],
    )
    def k(emb_hbm, out_hbm, buf0, buf1, rsem0, rsem1, wsem0, wsem1):
        bufs = (buf0, buf1)
        rsems = (rsem0, rsem1)
        wsems = (wsem0, wsem1)
        wid = lax.axis_index("s") * _NUM_CORES + lax.axis_index("c")
        base = row0 + wid * rows_per_w

        def read(c):
            b = c % 2
            return pltpu.async_copy(
                emb_hbm.at[pl.ds(base + c * chunk, chunk)], bufs[b], rsems[b])

        def write(c):
            b = c % 2
            return pltpu.async_copy(
                bufs[b], out_hbm.at[pl.ds(base + c * chunk, chunk)], wsems[b])

        reads = {0: read(0)}
        writes = {}
        for c in range(nchunk):
            if c + 1 < nchunk:
                if c - 1 >= 0:
                    writes.pop(c - 1).wait()
                reads[c + 1] = read(c + 1)
            reads.pop(c).wait()
            writes[c] = write(c)
        for w in writes.values():
            w.wait()

    return k


def _tc_body(carry_ref, emb_ref, out_ref):
    del carry_ref
    out_ref[...] = emb_ref[...]


@functools.lru_cache(maxsize=None)
def _make_tc_copy(n_rows: int, seq_len: int, dim: int):
    return pl.pallas_call(
        _tc_body,
        grid=(n_rows // _TC_BLOCK,),
        in_specs=[
            pl.BlockSpec(memory_space=pl.ANY),
            pl.BlockSpec((_TC_BLOCK, dim), lambda i: (i, 0)),
        ],
        out_specs=pl.BlockSpec((_TC_BLOCK, dim), lambda i: (i, 0)),
        out_shape=jax.ShapeDtypeStruct((seq_len, dim), jnp.float32),
        input_output_aliases={0: 0},
    )


def kernel(x, emb_weight):
    seq_len = x.shape[1]
    dim = emb_weight.shape[1]
    partial = _make_sc_copy(_TC_ROWS, seq_len, dim)(emb_weight)
    return _make_tc_copy(_TC_ROWS, seq_len, dim)(partial, emb_weight)
